# baseline (speedup 1.0000x reference)
import functools
import math

import jax
import jax.numpy as jnp
from jax import lax
from jax.experimental import pallas as pl
from jax.experimental.pallas import tpu as pltpu
from jax.experimental.pallas import tpu_sc as plsc

N_TOKEN = 1000000
D_MODEL = 64
SCALE = math.sqrt(D_MODEL)

NC, NS = 2, 16
NW = NC * NS
B_TOTAL = 4096 * 200
PER_W = B_TOTAL // NW
CHUNK = 128
N_CHUNKS = PER_W // CHUNK


def _emb_kernel(x_hbm, lut_hbm, out_hbm, idx_v, rows_v, gsem):
    wid = lax.axis_index("s") * NC + lax.axis_index("c")
    base = wid * PER_W
    pltpu.sync_copy(x_hbm.at[wid], idx_v)

    @pl.loop(0, N_CHUNKS)
    def _chunk(j):
        pltpu.async_copy(lut_hbm.at[idx_v.at[j]], rows_v, gsem).wait()

        @pl.loop(0, CHUNK)
        def _row(r):
            for c in range(D_MODEL // 16):
                sl = pl.ds(c * 16, 16)
                rows_v[r, sl] = rows_v[r, sl] * SCALE

        pltpu.sync_copy(rows_v, out_hbm.at[pl.ds(base + j * CHUNK, CHUNK)])


@jax.jit
def _emb(x3, lut):
    mesh = plsc.VectorSubcoreMesh(core_axis_name="c", subcore_axis_name="s")
    f = pl.kernel(
        _emb_kernel,
        out_type=jax.ShapeDtypeStruct((B_TOTAL, D_MODEL), jnp.float32),
        mesh=mesh,
        compiler_params=pltpu.CompilerParams(use_tc_tiling_on_sc=False),
        scratch_types=[
            pltpu.VMEM((N_CHUNKS, CHUNK), jnp.int32),
            pltpu.VMEM((CHUNK, D_MODEL), jnp.float32),
            pltpu.SemaphoreType.DMA,
        ],
    )
    return f(x3, lut)


def kernel(x, lut):
    b, t = x.shape
    x3 = x.reshape(NW, N_CHUNKS, CHUNK).astype(jnp.int32)
    out = _emb(x3, lut)
    return out.reshape(b, t, D_MODEL)


# speedup vs baseline: 1.2080x; 1.2080x over previous
import math

import jax
import jax.numpy as jnp
from jax import lax
from jax.experimental import pallas as pl
from jax.experimental.pallas import tpu as pltpu
from jax.experimental.pallas import tpu_sc as plsc

N_TOKEN = 1000000
D_MODEL = 64
SCALE = math.sqrt(D_MODEL)

NC, NS = 2, 16
NW = NC * NS
B_TOTAL = 4096 * 200
PER_W = B_TOTAL // NW
CHUNK = 128
N_CHUNKS = PER_W // CHUNK
NBUF = 4
LOOKAHEAD = 2


def _emb_kernel(x_hbm, lut_hbm, out_hbm, idx_v, rows, *sems):
    g_sems = sems[:NBUF]
    s_sems = sems[NBUF:]
    wid = lax.axis_index("s") * NC + lax.axis_index("c")
    base = wid * PER_W
    pltpu.sync_copy(x_hbm.at[wid], idx_v)

    def gather_start(j, b):
        pltpu.async_copy(lut_hbm.at[idx_v.at[j]], rows.at[b], g_sems[b])

    def gather_wait(j, b):
        pltpu.make_async_copy(lut_hbm.at[idx_v.at[j]], rows.at[b],
                              g_sems[b]).wait()

    def store_start(j, b):
        pltpu.async_copy(rows.at[b],
                         out_hbm.at[pl.ds(base + j * CHUNK, CHUNK)], s_sems[b])

    def store_wait(b):
        pltpu.make_async_copy(rows.at[b], out_hbm.at[pl.ds(base, CHUNK)],
                              s_sems[b]).wait()

    def scale(b):
        @pl.loop(0, CHUNK, unroll=4)
        def _row(r):
            for c in range(D_MODEL // 16):
                sl = pl.ds(c * 16, 16)
                rows[b, r, sl] = rows[b, r, sl] * SCALE

    def process(j, b, issue_j=None):
        if issue_j is not None:
            bb = (b + LOOKAHEAD) % NBUF
            if issue_j is not None and not isinstance(issue_j, int):
                store_wait(bb)
                gather_start(issue_j, bb)
            else:
                if issue_j >= NBUF:
                    store_wait(bb)
                gather_start(issue_j, bb)
        gather_wait(j, b)
        scale(b)
        store_start(j, b)

    for j in range(LOOKAHEAD):
        gather_start(j, j % NBUF)

    for b in range(NBUF):
        process(b, b, issue_j=b + LOOKAHEAD)

    @pl.loop(NBUF, N_CHUNKS - NBUF, step=NBUF)
    def _grp(j0):
        for b in range(NBUF):
            process(j0 + b, b, issue_j=j0 + b + LOOKAHEAD)

    for b in range(NBUF):
        j = N_CHUNKS - NBUF + b
        process(j, b,
                issue_j=(j + LOOKAHEAD) if j + LOOKAHEAD < N_CHUNKS else None)

    for b in range(NBUF):
        store_wait(b)


@jax.jit
def _emb(x3, lut):
    mesh = plsc.VectorSubcoreMesh(core_axis_name="c", subcore_axis_name="s")
    f = pl.kernel(
        _emb_kernel,
        out_type=jax.ShapeDtypeStruct((B_TOTAL, D_MODEL), jnp.float32),
        mesh=mesh,
        compiler_params=pltpu.CompilerParams(use_tc_tiling_on_sc=False),
        scratch_types=(
            [pltpu.VMEM((N_CHUNKS, CHUNK), jnp.int32),
             pltpu.VMEM((NBUF, CHUNK, D_MODEL), jnp.float32)]
            + [pltpu.SemaphoreType.DMA] * (2 * NBUF)
        ),
    )
    return f(x3, lut)


def kernel(x, lut):
    b, t = x.shape
    x3 = x.reshape(NW, N_CHUNKS, CHUNK).astype(jnp.int32)
    out = _emb(x3, lut)
    return out.reshape(b, t, D_MODEL)
